# Initial kernel scaffold; baseline (speedup 1.0000x reference)
import jax
import jax.numpy as jnp
from jax.experimental import pallas as pl


def kernel(inputs, token_table, position_table):
    raise NotImplementedError("write your pallas kernel here")


# speedup vs baseline: 1.8854x; 1.8854x over previous
import functools

import jax
import jax.numpy as jnp
from jax import lax
from jax.experimental import pallas as pl
from jax.experimental.pallas import tpu as pltpu
from jax.experimental.pallas import tpu_sc as plsc

SEQ = 200
DIM = 128
BATCH = 1024
ROWS = BATCH * SEQ
CHUNK = 128
NCHUNK = ROWS // CHUNK

_info = plsc.get_sparse_core_info()
_NC, _NS = _info.num_cores, _info.num_subcores
_NW = _NC * _NS
_CPW = NCHUNK // _NW


def _emb_kernel(idx_hbm, token_hbm, pos_hbm, out_hbm, idx_v, rows_v, pos_v, sem):
    wid = lax.axis_index("s") * _NC + lax.axis_index("c")
    c0 = wid * _CPW
    pltpu.sync_copy(idx_hbm.at[wid], idx_v)
    pltpu.sync_copy(pos_hbm, pos_v.at[pl.ds(0, SEQ)])
    pltpu.sync_copy(pos_hbm, pos_v.at[pl.ds(SEQ, SEQ)])

    def per_chunk(k, carry):
        pltpu.async_copy(token_hbm.at[idx_v.at[k]], rows_v, sem).wait()
        bm = lax.rem((c0 + k) * CHUNK, SEQ)

        def add_row(i, carry2):
            for j in range(DIM // 16):
                sl = pl.ds(j * 16, 16)
                rows_v[i, sl] = rows_v[i, sl] + pos_v[bm + i, sl]
            return carry2

        lax.fori_loop(0, CHUNK, add_row, 0)
        pltpu.sync_copy(rows_v, out_hbm.at[pl.ds((c0 + k) * CHUNK, CHUNK)])
        return carry

    lax.fori_loop(0, _CPW, per_chunk, 0)


@jax.jit
def kernel(inputs, token_table, position_table):
    idx = inputs.astype(jnp.int32).reshape(_NW, _CPW, CHUNK)
    run = functools.partial(
        pl.kernel,
        mesh=plsc.VectorSubcoreMesh(core_axis_name="c", subcore_axis_name="s"),
        out_type=jax.ShapeDtypeStruct((ROWS, DIM), jnp.float32),
        scratch_types=[
            pltpu.VMEM((_CPW, CHUNK), jnp.int32),
            pltpu.VMEM((CHUNK, DIM), jnp.float32),
            pltpu.VMEM((2 * SEQ, DIM), jnp.float32),
            pltpu.SemaphoreType.DMA,
        ],
    )(_emb_kernel)
    out = run(idx, token_table, position_table)
    return out.reshape(BATCH, SEQ, DIM)


# speedup vs baseline: 6.8402x; 3.6281x over previous
import functools

import jax
import jax.numpy as jnp
from jax import lax
from jax.experimental import pallas as pl
from jax.experimental.pallas import tpu as pltpu
from jax.experimental.pallas import tpu_sc as plsc

SEQ = 200
DIM = 128
BATCH = 1024
ROWS = BATCH * SEQ
CHUNK = 128
NCHUNK = ROWS // CHUNK

_info = plsc.get_sparse_core_info()
_NC, _NS = _info.num_cores, _info.num_subcores
_NW = _NC * _NS
_CPW = NCHUNK // _NW


def _emb_kernel(idx_hbm, token_hbm, pos_hbm, out_hbm,
                idx_v, rows_v, pos_sh, g0, g1, w0, w1):
    gsem = (g0, g1)
    wsem = (w0, w1)
    sub = lax.axis_index("s")
    wid = sub * _NC + lax.axis_index("c")
    c0 = wid * _CPW
    pltpu.sync_copy(idx_hbm.at[wid], idx_v)

    @pl.when(sub == 0)
    def _stage_pos():
        pltpu.sync_copy(pos_hbm, pos_sh.at[pl.ds(0, SEQ)])
        pltpu.sync_copy(pos_hbm, pos_sh.at[pl.ds(SEQ, SEQ)])

    plsc.subcore_barrier()

    def start_chunk(kk, b):
        bm = pl.multiple_of(lax.rem((c0 + kk) * CHUNK, SEQ), 8)
        pltpu.sync_copy(pos_sh.at[pl.ds(bm, CHUNK)], rows_v.at[b])
        pltpu.async_copy(token_hbm.at[idx_v.at[kk]], rows_v.at[b], gsem[b],
                         add=True)

    start_chunk(0, 0)

    def outer(k2, carry):
        for b in (0, 1):
            k = k2 * 2 + b
            nb = 1 - b

            @pl.when(k + 1 < _CPW)
            def _start_next():
                @pl.when(k >= 1)
                def _wait_prev_write():
                    pltpu.make_async_copy(
                        rows_v.at[nb], out_hbm.at[pl.ds(0, CHUNK)], wsem[nb]
                    ).wait()

                start_chunk(k + 1, nb)

            pltpu.make_async_copy(
                token_hbm.at[idx_v.at[k]], rows_v.at[b], gsem[b]
            ).wait()
            pltpu.async_copy(
                rows_v.at[b], out_hbm.at[pl.ds((c0 + k) * CHUNK, CHUNK)],
                wsem[b])
        return carry

    lax.fori_loop(0, _CPW // 2, outer, 0)
    pltpu.make_async_copy(rows_v.at[0], out_hbm.at[pl.ds(0, CHUNK)], w0).wait()
    pltpu.make_async_copy(rows_v.at[1], out_hbm.at[pl.ds(0, CHUNK)], w1).wait()


@jax.jit
def kernel(inputs, token_table, position_table):
    idx = inputs.astype(jnp.int32).reshape(_NW, _CPW, CHUNK)
    run = functools.partial(
        pl.kernel,
        mesh=plsc.VectorSubcoreMesh(core_axis_name="c", subcore_axis_name="s"),
        out_type=jax.ShapeDtypeStruct((ROWS, DIM), jnp.float32),
        scratch_types=[
            pltpu.VMEM((_CPW, CHUNK), jnp.int32),
            pltpu.VMEM((2, CHUNK, DIM), jnp.float32),
            pltpu.VMEM_SHARED((2 * SEQ, DIM), jnp.float32),
            pltpu.SemaphoreType.DMA,
            pltpu.SemaphoreType.DMA,
            pltpu.SemaphoreType.DMA,
            pltpu.SemaphoreType.DMA,
        ],
    )(_emb_kernel)
    out = run(idx, token_table, position_table)
    return out.reshape(BATCH, SEQ, DIM)


# speedup vs baseline: 7.6400x; 1.1169x over previous
import functools

import jax
import jax.numpy as jnp
from jax import lax
from jax.experimental import pallas as pl
from jax.experimental.pallas import tpu as pltpu
from jax.experimental.pallas import tpu_sc as plsc

SEQ = 200
DIM = 128
BATCH = 1024
ROWS = BATCH * SEQ
CHUNK = 128
NCHUNK = ROWS // CHUNK

_info = plsc.get_sparse_core_info()
_NC, _NS = _info.num_cores, _info.num_subcores
_NW = _NC * _NS
_CPW = NCHUNK // _NW


_NBUF = 3


def _emb_kernel(idx_hbm, token_hbm, pos_hbm, out_hbm,
                idx_v, rows_v, pos_sh, psem, gsem, wsem):
    sub = lax.axis_index("s")
    wid = sub * _NC + lax.axis_index("c")
    c0 = wid * _CPW
    pltpu.sync_copy(idx_hbm.at[wid], idx_v)

    @pl.when(sub == 0)
    def _stage_pos():
        pltpu.sync_copy(pos_hbm, pos_sh.at[pl.ds(0, SEQ)])
        pltpu.sync_copy(pos_hbm, pos_sh.at[pl.ds(SEQ, SEQ)])

    plsc.subcore_barrier()

    def prefill(kk, b):
        bm = pl.multiple_of(lax.rem((c0 + kk) * CHUNK, SEQ), 8)
        pltpu.async_copy(pos_sh.at[pl.ds(bm, CHUNK)], rows_v.at[b],
                         psem.at[b])

    def gather(kk, b):
        pltpu.make_async_copy(pos_sh.at[pl.ds(0, CHUNK)], rows_v.at[b],
                              psem.at[b]).wait()
        pltpu.async_copy(token_hbm.at[idx_v.at[kk]], rows_v.at[b],
                         gsem.at[b], add=True)

    def write(kk, b):
        pltpu.make_async_copy(token_hbm.at[idx_v.at[0]], rows_v.at[b],
                              gsem.at[b]).wait()
        pltpu.async_copy(rows_v.at[b], out_hbm.at[pl.ds((c0 + kk) * CHUNK,
                                                        CHUNK)], wsem.at[b])

    prefill(0, 0)
    prefill(1, 1)
    gather(0, 0)

    def body(j, carry):
        b = lax.rem(j, _NBUF)
        b1 = lax.rem(j + 1, _NBUF)
        b2 = lax.rem(j + 2, _NBUF)

        @pl.when(j + 2 < _CPW)
        def _start_prefill():
            @pl.when(j >= 1)
            def _wait_old_write():
                pltpu.make_async_copy(
                    rows_v.at[b2], out_hbm.at[pl.ds(0, CHUNK)], wsem.at[b2]
                ).wait()

            prefill(j + 2, b2)

        @pl.when(j + 1 < _CPW)
        def _start_gather():
            gather(j + 1, b1)

        write(j, b)
        return carry

    lax.fori_loop(0, _CPW, body, 0)
    for b in range(_NBUF):
        pltpu.make_async_copy(rows_v.at[b], out_hbm.at[pl.ds(0, CHUNK)],
                              wsem.at[b]).wait()


@jax.jit
def kernel(inputs, token_table, position_table):
    idx = inputs.astype(jnp.int32).reshape(_NW, _CPW, CHUNK)
    run = functools.partial(
        pl.kernel,
        mesh=plsc.VectorSubcoreMesh(core_axis_name="c", subcore_axis_name="s"),
        out_type=jax.ShapeDtypeStruct((ROWS, DIM), jnp.float32),
        scratch_types=[
            pltpu.VMEM((_CPW, CHUNK), jnp.int32),
            pltpu.VMEM((_NBUF, CHUNK, DIM), jnp.float32),
            pltpu.VMEM_SHARED((2 * SEQ, DIM), jnp.float32),
            pltpu.SemaphoreType.DMA((_NBUF,)),
            pltpu.SemaphoreType.DMA((_NBUF,)),
            pltpu.SemaphoreType.DMA((_NBUF,)),
        ],
    )(_emb_kernel)
    out = run(idx, token_table, position_table)
    return out.reshape(BATCH, SEQ, DIM)
